# Initial kernel scaffold; baseline (speedup 1.0000x reference)
import functools

import jax
import jax.numpy as jnp
from jax import lax
from jax.experimental import pallas as pl
from jax.experimental.pallas import tpu as pltpu
from jax.experimental.pallas import tpu_sc as plsc

N = 16384
INPUT_DIM = 40960
INTER_DIM = 641
OUT_DIM = 32
NNZ = 524288

NC = 2
NS = 16
LANES = 16
NW = NC * NS
CHUNK = NNZ // NW
BLK = 128
NBLK = CHUNK // BLK
GROUPS = BLK // LANES
ROWS_PER_TEC = N // NS


def _sc_body(bi_hbm, ai_hbm, val_hbm, w_hbm, z_hbm, out_hbm,
             acc, w_v, bi_v, ai_v, val_v, rows_v):
    cid = lax.axis_index("c")
    sid = lax.axis_index("s")
    wid = sid * NC + cid

    base = wid * NBLK
    pltpu.sync_copy(bi_hbm.at[pl.ds(base, NBLK)], bi_v)
    pltpu.sync_copy(ai_hbm.at[pl.ds(base, NBLK)], ai_v)
    pltpu.sync_copy(val_hbm.at[pl.ds(base, NBLK)], val_v)
    pltpu.sync_copy(w_hbm, w_v)
    pltpu.sync_copy(z_hbm.at[pl.ds(sid * ROWS_PER_TEC, ROWS_PER_TEC)],
                    acc.at[pl.ds(sid * ROWS_PER_TEC, ROWS_PER_TEC)])
    plsc.subcore_barrier()

    iota = lax.iota(jnp.int32, LANES)

    def block(j, carry):
        for g in range(GROUPS):
            sl = pl.ds(g * LANES, LANES)
            a16 = ai_v[j, sl]
            v16 = val_v[j, sl]
            bucket = lax.rem(a16, INTER_DIM)
            widx0 = bucket * OUT_DIM
            rid = g * LANES + iota
            for c in range(OUT_DIM):
                wc = plsc.load_gather(w_v, [widx0 + c])
                cvec = jnp.full((LANES,), c, jnp.int32)
                plsc.store_scatter(rows_v, [rid, cvec], v16 * wc)
        pltpu.sync_copy(rows_v, acc.at[bi_v.at[j]], add=True)
        return carry

    lax.fori_loop(0, NBLK, block, 0)
    plsc.subcore_barrier()
    pltpu.sync_copy(acc.at[pl.ds(sid * ROWS_PER_TEC, ROWS_PER_TEC)],
                    out_hbm.at[cid, pl.ds(sid * ROWS_PER_TEC, ROWS_PER_TEC)])


@jax.jit
def _sc_call(bi, ai, val, wflat, zeros):
    mesh = plsc.VectorSubcoreMesh(core_axis_name="c", subcore_axis_name="s")
    return pl.kernel(
        _sc_body,
        out_type=jax.ShapeDtypeStruct((NC, N, OUT_DIM), jnp.float32),
        mesh=mesh,
        scratch_types=[
            pltpu.VMEM_SHARED((N, OUT_DIM), jnp.float32),
            pltpu.VMEM((INTER_DIM * OUT_DIM,), jnp.float32),
            pltpu.VMEM((NBLK, BLK), jnp.int32),
            pltpu.VMEM((NBLK, BLK), jnp.int32),
            pltpu.VMEM((NBLK, BLK), jnp.float32),
            pltpu.VMEM((BLK, OUT_DIM), jnp.float32),
        ],
    )(bi, ai, val, wflat, zeros)


def _add_body(p_ref, o_ref):
    o_ref[...] = p_ref[0] + p_ref[1]


@jax.jit
def _tc_add(partials):
    return pl.pallas_call(
        _add_body,
        out_shape=jax.ShapeDtypeStruct((N, OUT_DIM), jnp.float32),
    )(partials)


def kernel(batch_idx, active_idx, values, f, weights):
    del f
    bi = batch_idx.astype(jnp.int32).reshape(NNZ // BLK, BLK)
    ai = active_idx.astype(jnp.int32).reshape(NNZ // BLK, BLK)
    val = values.reshape(NNZ // BLK, BLK)
    wflat = weights.reshape(-1)
    zeros = jnp.zeros((N, OUT_DIM), jnp.float32)
    partials = _sc_call(bi, ai, val, wflat, zeros)
    return _tc_add(partials)


# speedup vs baseline: 4.0479x; 4.0479x over previous
import functools

import jax
import jax.numpy as jnp
from jax import lax
from jax.experimental import pallas as pl
from jax.experimental.pallas import tpu as pltpu
from jax.experimental.pallas import tpu_sc as plsc

N = 16384
INPUT_DIM = 40960
INTER_DIM = 641
OUT_DIM = 32
NNZ = 524288

NC = 2
NS = 16
LANES = 16
NW = NC * NS
CHUNK = NNZ // NW
BLK = 128
NBLK = CHUNK // BLK
GROUPS = BLK // LANES
PACK = 128 // OUT_DIM
NPACK = N // PACK
ROWS_PER_TEC = NPACK // NS


def _sc_body(bi_hbm, ai_hbm, val_hbm, w_hbm, z_hbm, out_hbm,
             acc, w_v, bi_v, ai_v, val_v, rows_v, bidx_v):
    cid = lax.axis_index("c")
    sid = lax.axis_index("s")
    wid = sid * NC + cid

    base = wid * NBLK
    pltpu.sync_copy(bi_hbm.at[pl.ds(base, NBLK)], bi_v)
    pltpu.sync_copy(ai_hbm.at[pl.ds(base, NBLK)], ai_v)
    pltpu.sync_copy(val_hbm.at[pl.ds(base, NBLK)], val_v)
    pltpu.sync_copy(w_hbm, w_v)
    pltpu.sync_copy(z_hbm.at[pl.ds(sid * ROWS_PER_TEC, ROWS_PER_TEC)],
                    acc.at[pl.ds(sid * ROWS_PER_TEC, ROWS_PER_TEC)])
    plsc.subcore_barrier()

    iota = lax.iota(jnp.int32, LANES)
    zero16 = jnp.zeros((LANES,), jnp.float32)

    def block(j, carry):
        for r in range(BLK):
            for q in range(128 // LANES):
                rows_v[r, pl.ds(q * LANES, LANES)] = zero16
        for g in range(GROUPS):
            sl = pl.ds(g * LANES, LANES)
            b16 = bi_v[j, sl]
            a16 = ai_v[j, sl]
            v16 = val_v[j, sl]
            bidx_v[sl] = lax.shift_right_logical(b16, 2)
            colbase = lax.shift_left(lax.bitwise_and(b16, PACK - 1), 5)
            bucket = lax.rem(a16, INTER_DIM)
            widx0 = bucket * OUT_DIM
            rid = g * LANES + iota
            for c in range(OUT_DIM):
                wc = plsc.load_gather(w_v, [widx0 + c])
                plsc.store_scatter(rows_v, [rid, colbase + c], v16 * wc)
        pltpu.sync_copy(rows_v, acc.at[bidx_v], add=True)
        return carry

    lax.fori_loop(0, NBLK, block, 0)
    plsc.subcore_barrier()
    pltpu.sync_copy(acc.at[pl.ds(sid * ROWS_PER_TEC, ROWS_PER_TEC)],
                    out_hbm.at[cid, pl.ds(sid * ROWS_PER_TEC, ROWS_PER_TEC)])


@jax.jit
def _sc_call(bi, ai, val, wflat, zeros):
    mesh = plsc.VectorSubcoreMesh(core_axis_name="c", subcore_axis_name="s")
    return pl.kernel(
        _sc_body,
        out_type=jax.ShapeDtypeStruct((NC, NPACK, 128), jnp.float32),
        mesh=mesh,
        scratch_types=[
            pltpu.VMEM_SHARED((NPACK, 128), jnp.float32),
            pltpu.VMEM((INTER_DIM * OUT_DIM,), jnp.float32),
            pltpu.VMEM((NBLK, BLK), jnp.int32),
            pltpu.VMEM((NBLK, BLK), jnp.int32),
            pltpu.VMEM((NBLK, BLK), jnp.float32),
            pltpu.VMEM((BLK, 128), jnp.float32),
            pltpu.VMEM((BLK,), jnp.int32),
        ],
        compiler_params=pltpu.CompilerParams(needs_layout_passes=False),
    )(bi, ai, val, wflat, zeros)


def _add_body(p_ref, o_ref):
    o_ref[...] = p_ref[0] + p_ref[1]


@jax.jit
def _tc_add(partials):
    summed = pl.pallas_call(
        _add_body,
        out_shape=jax.ShapeDtypeStruct((NPACK, 128), jnp.float32),
    )(partials)
    return summed.reshape(N, OUT_DIM)


def kernel(batch_idx, active_idx, values, f, weights):
    del f
    bi = batch_idx.astype(jnp.int32).reshape(NNZ // BLK, BLK)
    ai = active_idx.astype(jnp.int32).reshape(NNZ // BLK, BLK)
    val = values.reshape(NNZ // BLK, BLK)
    wflat = weights.reshape(-1)
    zeros = jnp.zeros((NPACK, 128), jnp.float32)
    partials = _sc_call(bi, ai, val, wflat, zeros)
    return _tc_add(partials)


# speedup vs baseline: 5.2442x; 1.2955x over previous
import functools

import jax
import jax.numpy as jnp
from jax import lax
from jax.experimental import pallas as pl
from jax.experimental.pallas import tpu as pltpu
from jax.experimental.pallas import tpu_sc as plsc

N = 16384
INPUT_DIM = 40960
INTER_DIM = 641
OUT_DIM = 32
NNZ = 524288

NC = 2
NS = 16
LANES = 16
NW = NC * NS
CHUNK = NNZ // NW
BLK = 64
NBLK = CHUNK // BLK
GROUPS = BLK // LANES
CROWS = CHUNK // 128
PACK = 128 // OUT_DIM
NPACK = N // PACK
ROWS_PER_TEC = NPACK // NS


def _sc_body(bi_hbm, ai_hbm, val_hbm, w_hbm, z_hbm, out_hbm,
             acc, w_v, bi_v, ai_v, val_v, rows3, bidx3, sems):
    cid = lax.axis_index("c")
    sid = lax.axis_index("s")
    wid = sid * NC + cid

    base = wid * CROWS
    pltpu.sync_copy(bi_hbm.at[pl.ds(base, CROWS)], bi_v)
    pltpu.sync_copy(ai_hbm.at[pl.ds(base, CROWS)], ai_v)
    pltpu.sync_copy(val_hbm.at[pl.ds(base, CROWS)], val_v)
    pltpu.sync_copy(w_hbm, w_v)
    pltpu.sync_copy(z_hbm.at[pl.ds(sid * ROWS_PER_TEC, ROWS_PER_TEC)],
                    acc.at[pl.ds(sid * ROWS_PER_TEC, ROWS_PER_TEC)])
    plsc.subcore_barrier()

    iota = lax.iota(jnp.int32, LANES)
    zero16 = jnp.zeros((LANES,), jnp.float32)

    def build(j, p, pvec):
        def zrow(z, carry):
            for rr in range(8):
                for q in range(128 // LANES):
                    rows3[p, z * 8 + rr, pl.ds(q * LANES, LANES)] = zero16
            return carry
        lax.fori_loop(0, 8, zrow, 0)
        crow = lax.shift_right_logical(j, 1)
        cbase = lax.shift_left(lax.bitwise_and(j, 1), 6)
        for g in range(GROUPS):
            sl = pl.ds(cbase + g * LANES, LANES)
            b16 = bi_v[crow, sl]
            a16 = ai_v[crow, sl]
            v16 = val_v[crow, sl]
            bidx3[p, pl.ds(g * LANES, LANES)] = lax.shift_right_logical(b16, 2)
            colbase = lax.shift_left(lax.bitwise_and(b16, PACK - 1), 5)
            bucket = lax.rem(a16, INTER_DIM)
            widx0 = bucket * OUT_DIM
            rid = g * LANES + iota
            for c in range(OUT_DIM):
                wc = plsc.load_gather(w_v, [widx0 + c])
                plsc.store_scatter(rows3, [pvec, rid, colbase + c], v16 * wc)

    def body(j, carry):
        p = lax.rem(j, 2)

        @pl.when(j >= 2)
        def _():
            pltpu.make_async_copy(
                rows3.at[p], acc.at[bidx3.at[p]], sems.at[p]).wait()

        @pl.when(j < NBLK)
        def _():
            build(j, p, jnp.full((LANES,), p, jnp.int32))
            pltpu.make_async_copy(
                rows3.at[p], acc.at[bidx3.at[p]], sems.at[p]).start(add=True)
        return carry

    lax.fori_loop(0, NBLK + 2, body, 0)
    plsc.subcore_barrier()
    pltpu.sync_copy(acc.at[pl.ds(sid * ROWS_PER_TEC, ROWS_PER_TEC)],
                    out_hbm.at[cid, pl.ds(sid * ROWS_PER_TEC, ROWS_PER_TEC)])


@jax.jit
def _sc_call(bi, ai, val, wflat, zeros):
    mesh = plsc.VectorSubcoreMesh(core_axis_name="c", subcore_axis_name="s")
    return pl.kernel(
        _sc_body,
        out_type=jax.ShapeDtypeStruct((NC, NPACK, 128), jnp.float32),
        mesh=mesh,
        scratch_types=[
            pltpu.VMEM_SHARED((NPACK, 128), jnp.float32),
            pltpu.VMEM((INTER_DIM * OUT_DIM,), jnp.float32),
            pltpu.VMEM((CROWS, 128), jnp.int32),
            pltpu.VMEM((CROWS, 128), jnp.int32),
            pltpu.VMEM((CROWS, 128), jnp.float32),
            pltpu.VMEM((2, BLK, 128), jnp.float32),
            pltpu.VMEM((2, BLK), jnp.int32),
            pltpu.SemaphoreType.DMA((2,)),
        ],
        compiler_params=pltpu.CompilerParams(needs_layout_passes=False),
    )(bi, ai, val, wflat, zeros)


def _add_body(p_ref, o_ref):
    o_ref[...] = p_ref[0] + p_ref[1]


@jax.jit
def _tc_add(partials):
    summed = pl.pallas_call(
        _add_body,
        out_shape=jax.ShapeDtypeStruct((NPACK, 128), jnp.float32),
    )(partials)
    return summed.reshape(N, OUT_DIM)


def kernel(batch_idx, active_idx, values, f, weights):
    del f
    bi = batch_idx.astype(jnp.int32).reshape(NNZ // 128, 128)
    ai = active_idx.astype(jnp.int32).reshape(NNZ // 128, 128)
    val = values.reshape(NNZ // 128, 128)
    wflat = weights.reshape(-1)
    zeros = jnp.zeros((NPACK, 128), jnp.float32)
    partials = _sc_call(bi, ai, val, wflat, zeros)
    return _tc_add(partials)


# speedup vs baseline: 7.1567x; 1.3647x over previous
import functools

import jax
import jax.numpy as jnp
from jax import lax
from jax.experimental import pallas as pl
from jax.experimental.pallas import tpu as pltpu
from jax.experimental.pallas import tpu_sc as plsc

N = 16384
INPUT_DIM = 40960
INTER_DIM = 641
OUT_DIM = 32
NNZ = 524288

NC = 2
NS = 16
LANES = 16
NW = NC * NS
CHUNK = NNZ // NW
BLK = 64
NBLK = CHUNK // BLK
GROUPS = BLK // LANES
CROWS = CHUNK // 128
PACK = 128 // OUT_DIM
NPACK = N // PACK
ROWS_PER_TEC = NPACK // NS


def _sc_body(bi_hbm, ai_hbm, val_hbm, w_hbm, z_hbm, out_hbm,
             acc, w_v, bi_v, ai_v, val_v, rows3, bidx3, sems):
    cid = lax.axis_index("c")
    sid = lax.axis_index("s")
    wid = sid * NC + cid

    base = wid * CROWS
    pltpu.sync_copy(bi_hbm.at[pl.ds(base, CROWS)], bi_v)
    pltpu.sync_copy(ai_hbm.at[pl.ds(base, CROWS)], ai_v)
    pltpu.sync_copy(val_hbm.at[pl.ds(base, CROWS)], val_v)
    pltpu.sync_copy(w_hbm, w_v)
    pltpu.sync_copy(z_hbm.at[pl.ds(sid * ROWS_PER_TEC, ROWS_PER_TEC)],
                    acc.at[pl.ds(sid * ROWS_PER_TEC, ROWS_PER_TEC)])
    plsc.subcore_barrier()

    iota = lax.iota(jnp.int32, LANES)
    zero16 = jnp.zeros((LANES,), jnp.float32)

    def build(j, p, pvec):
        def zrow(z, carry):
            for rr in range(8):
                for q in range(128 // LANES):
                    rows3[p, z * 8 + rr, pl.ds(q * LANES, LANES)] = zero16
            return carry
        lax.fori_loop(0, 8, zrow, 0)
        crow = lax.shift_right_logical(j, 1)
        cbase = lax.shift_left(lax.bitwise_and(j, 1), 6)
        zero16i = jnp.zeros((LANES,), jnp.int32)
        for g in range(GROUPS):
            sl = pl.ds(cbase + g * LANES, LANES)
            b16 = bi_v[crow, sl]
            a16 = ai_v[crow, sl]
            v16 = val_v[crow, sl]
            bidx3[p, pl.ds(g * LANES, LANES)] = lax.shift_right_logical(b16, 2)
            colbase = lax.shift_left(lax.bitwise_and(b16, PACK - 1), 5)
            bucket = lax.rem(a16, INTER_DIM)
            widx0 = bucket * OUT_DIM
            rid = g * LANES + iota
            sbase = ((pvec * BLK + rid) * 128) + colbase
            prods = [plsc.load_gather(w_v, [widx0 + c]) * v16
                     for c in range(OUT_DIM)]
            for c in range(OUT_DIM):
                plsc.store_scatter(rows3, [zero16i, zero16i, sbase + c],
                                   prods[c])

    def body(j, carry):
        p = lax.rem(j, 2)

        @pl.when(j >= 2)
        def _():
            pltpu.make_async_copy(
                rows3.at[p], acc.at[bidx3.at[p]], sems.at[p]).wait()

        @pl.when(j < NBLK)
        def _():
            build(j, p, jnp.full((LANES,), p, jnp.int32))
            pltpu.make_async_copy(
                rows3.at[p], acc.at[bidx3.at[p]], sems.at[p]).start(add=True)
        return carry

    lax.fori_loop(0, NBLK + 2, body, 0)
    plsc.subcore_barrier()
    pltpu.sync_copy(acc.at[pl.ds(sid * ROWS_PER_TEC, ROWS_PER_TEC)],
                    out_hbm.at[cid, pl.ds(sid * ROWS_PER_TEC, ROWS_PER_TEC)])


@jax.jit
def _sc_call(bi, ai, val, wflat, zeros):
    mesh = plsc.VectorSubcoreMesh(core_axis_name="c", subcore_axis_name="s")
    return pl.kernel(
        _sc_body,
        out_type=jax.ShapeDtypeStruct((NC, NPACK, 128), jnp.float32),
        mesh=mesh,
        scratch_types=[
            pltpu.VMEM_SHARED((NPACK, 128), jnp.float32),
            pltpu.VMEM((INTER_DIM * OUT_DIM,), jnp.float32),
            pltpu.VMEM((CROWS, 128), jnp.int32),
            pltpu.VMEM((CROWS, 128), jnp.int32),
            pltpu.VMEM((CROWS, 128), jnp.float32),
            pltpu.VMEM((2, BLK, 128), jnp.float32),
            pltpu.VMEM((2, BLK), jnp.int32),
            pltpu.SemaphoreType.DMA((2,)),
        ],
        compiler_params=pltpu.CompilerParams(needs_layout_passes=False),
    )(bi, ai, val, wflat, zeros)


def _add_body(p_ref, o_ref):
    o_ref[...] = p_ref[0] + p_ref[1]


@jax.jit
def _tc_add(partials):
    summed = pl.pallas_call(
        _add_body,
        out_shape=jax.ShapeDtypeStruct((NPACK, 128), jnp.float32),
    )(partials)
    return summed.reshape(N, OUT_DIM)


def kernel(batch_idx, active_idx, values, f, weights):
    del f
    bi = batch_idx.astype(jnp.int32).reshape(NNZ // 128, 128)
    ai = active_idx.astype(jnp.int32).reshape(NNZ // 128, 128)
    val = values.reshape(NNZ // 128, 128)
    wflat = weights.reshape(-1)
    zeros = jnp.zeros((NPACK, 128), jnp.float32)
    partials = _sc_call(bi, ai, val, wflat, zeros)
    return _tc_add(partials)


# speedup vs baseline: 7.6563x; 1.0698x over previous
import functools

import jax
import jax.numpy as jnp
from jax import lax
from jax.experimental import pallas as pl
from jax.experimental.pallas import tpu as pltpu
from jax.experimental.pallas import tpu_sc as plsc

N = 16384
INPUT_DIM = 40960
INTER_DIM = 641
OUT_DIM = 32
NNZ = 524288

NC = 2
NS = 16
LANES = 16
NW = NC * NS
CHUNK = NNZ // NW
BLK = 64
NBLK = CHUNK // BLK
GROUPS = BLK // LANES
CROWS = CHUNK // 128
PACK = 128 // OUT_DIM
NPACK = N // PACK
ROWS_PER_TEC = NPACK // NS


def _sc_body(bi_hbm, ai_hbm, val_hbm, w_hbm, z_hbm, out_hbm,
             acc, w_v, bi_v, ai_v, val_v, rows3, bidx3, sems):
    cid = lax.axis_index("c")
    sid = lax.axis_index("s")
    wid = sid * NC + cid

    base = wid * CROWS
    pltpu.sync_copy(bi_hbm.at[pl.ds(base, CROWS)], bi_v)
    pltpu.sync_copy(ai_hbm.at[pl.ds(base, CROWS)], ai_v)
    pltpu.sync_copy(val_hbm.at[pl.ds(base, CROWS)], val_v)
    pltpu.sync_copy(w_hbm, w_v)
    pltpu.sync_copy(z_hbm.at[pl.ds(sid * ROWS_PER_TEC, ROWS_PER_TEC)],
                    acc.at[pl.ds(sid * ROWS_PER_TEC, ROWS_PER_TEC)])
    plsc.subcore_barrier()

    iota = lax.iota(jnp.int32, LANES)
    zero16 = jnp.zeros((LANES,), jnp.float32)

    def build(j, p, pvec):
        def zrow(z, carry):
            for rr in range(8):
                for q in range(128 // LANES):
                    rows3[p, z * 8 + rr, pl.ds(q * LANES, LANES)] = zero16
            return carry
        lax.fori_loop(0, 8, zrow, 0)
        crow = lax.shift_right_logical(j, 1)
        cbase = lax.shift_left(lax.bitwise_and(j, 1), 6)
        zero16i = jnp.zeros((LANES,), jnp.int32)
        for g in range(GROUPS):
            sl = pl.ds(cbase + g * LANES, LANES)
            b16 = bi_v[crow, sl]
            a16 = ai_v[crow, sl]
            v16 = val_v[crow, sl]
            bidx3[p, pl.ds(g * LANES, LANES)] = lax.shift_right_logical(b16, 2)
            colbase = lax.shift_left(lax.bitwise_and(b16, PACK - 1), 5)
            q = lax.shift_right_logical(a16 * 52348, 25)
            bucket = a16 - q * INTER_DIM
            widx0 = bucket * OUT_DIM
            rid = g * LANES + iota
            sbase = ((pvec * BLK + rid) * 128) + colbase
            for h in range(2):
                half = OUT_DIM // 2
                prods = [plsc.load_gather(w_v, [widx0 + (h * half + c)]) * v16
                         for c in range(half)]
                for c in range(half):
                    plsc.store_scatter(
                        rows3, [zero16i, zero16i, sbase + (h * half + c)],
                        prods[c])

    def body(j, carry):
        p = lax.rem(j, 2)

        @pl.when(j >= 2)
        def _():
            pltpu.make_async_copy(
                rows3.at[p], acc.at[bidx3.at[p]], sems.at[p]).wait()

        @pl.when(j < NBLK)
        def _():
            build(j, p, jnp.full((LANES,), p, jnp.int32))
            pltpu.make_async_copy(
                rows3.at[p], acc.at[bidx3.at[p]], sems.at[p]).start(add=True)
        return carry

    lax.fori_loop(0, NBLK + 2, body, 0)
    plsc.subcore_barrier()
    pltpu.sync_copy(acc.at[pl.ds(sid * ROWS_PER_TEC, ROWS_PER_TEC)],
                    out_hbm.at[cid, pl.ds(sid * ROWS_PER_TEC, ROWS_PER_TEC)])


@jax.jit
def _sc_call(bi, ai, val, wflat, zeros):
    mesh = plsc.VectorSubcoreMesh(core_axis_name="c", subcore_axis_name="s")
    return pl.kernel(
        _sc_body,
        out_type=jax.ShapeDtypeStruct((NC, NPACK, 128), jnp.float32),
        mesh=mesh,
        scratch_types=[
            pltpu.VMEM_SHARED((NPACK, 128), jnp.float32),
            pltpu.VMEM((INTER_DIM * OUT_DIM,), jnp.float32),
            pltpu.VMEM((CROWS, 128), jnp.int32),
            pltpu.VMEM((CROWS, 128), jnp.int32),
            pltpu.VMEM((CROWS, 128), jnp.float32),
            pltpu.VMEM((2, BLK, 128), jnp.float32),
            pltpu.VMEM((2, BLK), jnp.int32),
            pltpu.SemaphoreType.DMA((2,)),
        ],
        compiler_params=pltpu.CompilerParams(needs_layout_passes=False),
    )(bi, ai, val, wflat, zeros)


def _add_body(p_ref, o_ref):
    o_ref[...] = p_ref[0] + p_ref[1]


@jax.jit
def _tc_add(partials):
    summed = pl.pallas_call(
        _add_body,
        out_shape=jax.ShapeDtypeStruct((NPACK, 128), jnp.float32),
    )(partials)
    return summed.reshape(N, OUT_DIM)


def kernel(batch_idx, active_idx, values, f, weights):
    del f
    bi = batch_idx.astype(jnp.int32).reshape(NNZ // 128, 128)
    ai = active_idx.astype(jnp.int32).reshape(NNZ // 128, 128)
    val = values.reshape(NNZ // 128, 128)
    wflat = weights.reshape(-1)
    zeros = jnp.zeros((NPACK, 128), jnp.float32)
    partials = _sc_call(bi, ai, val, wflat, zeros)
    return _tc_add(partials)


# speedup vs baseline: 7.8413x; 1.0242x over previous
import functools

import jax
import jax.numpy as jnp
from jax import lax
from jax.experimental import pallas as pl
from jax.experimental.pallas import tpu as pltpu
from jax.experimental.pallas import tpu_sc as plsc

N = 16384
INPUT_DIM = 40960
INTER_DIM = 641
OUT_DIM = 32
NNZ = 524288

NC = 2
NS = 16
LANES = 16
NW = NC * NS
CHUNK = NNZ // NW
BLK = 64
NBLK = CHUNK // BLK
GROUPS = BLK // LANES
CROWS = CHUNK // 128
PACK = 128 // OUT_DIM
NPACK = N // PACK
ROWS_PER_TEC = NPACK // NS


def _sc_body(bi_hbm, ai_hbm, val_hbm, w_hbm, z_hbm, out_hbm,
             acc, w_v, bi_v, ai_v, val_v, rows3, bidx3, sems):
    cid = lax.axis_index("c")
    sid = lax.axis_index("s")
    wid = sid * NC + cid

    base = wid * CROWS
    pltpu.sync_copy(bi_hbm.at[pl.ds(base, CROWS)], bi_v)
    pltpu.sync_copy(ai_hbm.at[pl.ds(base, CROWS)], ai_v)
    pltpu.sync_copy(val_hbm.at[pl.ds(base, CROWS)], val_v)
    pltpu.sync_copy(w_hbm, w_v)
    pltpu.sync_copy(z_hbm.at[pl.ds(sid * ROWS_PER_TEC, ROWS_PER_TEC)],
                    acc.at[pl.ds(sid * ROWS_PER_TEC, ROWS_PER_TEC)])
    plsc.subcore_barrier()

    iota = lax.iota(jnp.int32, LANES)
    zero16 = jnp.zeros((LANES,), jnp.float32)

    def build(j, p, pvec):
        def zrow(z, carry):
            for rr in range(8):
                for q in range(128 // LANES):
                    rows3[p, z * 8 + rr, pl.ds(q * LANES, LANES)] = zero16
            return carry
        lax.fori_loop(0, 8, zrow, 0)
        crow = lax.shift_right_logical(j, 1)
        cbase = lax.shift_left(lax.bitwise_and(j, 1), 6)
        zero16i = jnp.zeros((LANES,), jnp.int32)
        for g in range(GROUPS):
            sl = pl.ds(cbase + g * LANES, LANES)
            b16 = bi_v[crow, sl]
            a16 = ai_v[crow, sl]
            v16 = val_v[crow, sl]
            bidx3[p, pl.ds(g * LANES, LANES)] = lax.shift_right_logical(b16, 2)
            colbase = lax.shift_left(lax.bitwise_and(b16, PACK - 1), 5)
            q = lax.shift_right_logical(a16 * 52348, 25)
            bucket = a16 - q * INTER_DIM
            widx0 = bucket * OUT_DIM
            rid = g * LANES + iota
            sbase = ((pvec * BLK + rid) * 128) + colbase
            for h in range(4):
                half = OUT_DIM // 4
                prods = [plsc.load_gather(w_v, [widx0 + (h * half + c)]) * v16
                         for c in range(half)]
                for c in range(half):
                    plsc.store_scatter(
                        rows3, [zero16i, zero16i, sbase + (h * half + c)],
                        prods[c])

    def body(j, carry):
        p = lax.rem(j, 2)

        @pl.when(j >= 2)
        def _():
            pltpu.make_async_copy(
                rows3.at[p], acc.at[bidx3.at[p]], sems.at[p]).wait()

        @pl.when(j < NBLK)
        def _():
            build(j, p, jnp.full((LANES,), p, jnp.int32))
            pltpu.make_async_copy(
                rows3.at[p], acc.at[bidx3.at[p]], sems.at[p]).start(add=True)
        return carry

    lax.fori_loop(0, NBLK + 2, body, 0)
    plsc.subcore_barrier()
    pltpu.sync_copy(acc.at[pl.ds(sid * ROWS_PER_TEC, ROWS_PER_TEC)],
                    out_hbm.at[cid, pl.ds(sid * ROWS_PER_TEC, ROWS_PER_TEC)])


@jax.jit
def _sc_call(bi, ai, val, wflat, zeros):
    mesh = plsc.VectorSubcoreMesh(core_axis_name="c", subcore_axis_name="s")
    return pl.kernel(
        _sc_body,
        out_type=jax.ShapeDtypeStruct((NC, NPACK, 128), jnp.float32),
        mesh=mesh,
        scratch_types=[
            pltpu.VMEM_SHARED((NPACK, 128), jnp.float32),
            pltpu.VMEM((INTER_DIM * OUT_DIM,), jnp.float32),
            pltpu.VMEM((CROWS, 128), jnp.int32),
            pltpu.VMEM((CROWS, 128), jnp.int32),
            pltpu.VMEM((CROWS, 128), jnp.float32),
            pltpu.VMEM((2, BLK, 128), jnp.float32),
            pltpu.VMEM((2, BLK), jnp.int32),
            pltpu.SemaphoreType.DMA((2,)),
        ],
        compiler_params=pltpu.CompilerParams(needs_layout_passes=False),
    )(bi, ai, val, wflat, zeros)


def _add_body(p_ref, o_ref):
    o_ref[...] = p_ref[0] + p_ref[1]


@jax.jit
def _tc_add(partials):
    summed = pl.pallas_call(
        _add_body,
        out_shape=jax.ShapeDtypeStruct((NPACK, 128), jnp.float32),
    )(partials)
    return summed.reshape(N, OUT_DIM)


def kernel(batch_idx, active_idx, values, f, weights):
    del f
    bi = batch_idx.astype(jnp.int32).reshape(NNZ // 128, 128)
    ai = active_idx.astype(jnp.int32).reshape(NNZ // 128, 128)
    val = values.reshape(NNZ // 128, 128)
    wflat = weights.reshape(-1)
    zeros = jnp.zeros((NPACK, 128), jnp.float32)
    partials = _sc_call(bi, ai, val, wflat, zeros)
    return _tc_add(partials)


# speedup vs baseline: 19.9095x; 2.5391x over previous
import functools

import jax
import jax.numpy as jnp
from jax import lax
from jax.experimental import pallas as pl
from jax.experimental.pallas import tpu as pltpu
from jax.experimental.pallas import tpu_sc as plsc

N = 16384
INPUT_DIM = 40960
INTER_DIM = 641
OUT_DIM = 32
NNZ = 524288

NC = 2
NS = 16
LANES = 16
NW = NC * NS
CHUNK = NNZ // NW
BLK = 64
NBLK = CHUNK // BLK
GROUPS = BLK // LANES
CROWS = CHUNK // 128
PACK = 128 // OUT_DIM
NPACK = N // PACK
ROWS_PER_TEC = NPACK // NS


def _sc_body(bi_hbm, ai_hbm, val_hbm, w_hbm, z_hbm, out_hbm,
             acc, w_v, bi_v, ai_v, val_v, rows3, bidx3, sems):
    cid = lax.axis_index("c")
    sid = lax.axis_index("s")
    wid = sid * NC + cid

    base = wid * CROWS
    pltpu.sync_copy(bi_hbm.at[pl.ds(base, CROWS)], bi_v)
    pltpu.sync_copy(ai_hbm.at[pl.ds(base, CROWS)], ai_v)
    pltpu.sync_copy(val_hbm.at[pl.ds(base, CROWS)], val_v)
    pltpu.sync_copy(w_hbm, w_v)
    pltpu.sync_copy(z_hbm.at[pl.ds(sid * ROWS_PER_TEC, ROWS_PER_TEC)],
                    acc.at[pl.ds(sid * ROWS_PER_TEC, ROWS_PER_TEC)])
    plsc.subcore_barrier()

    iota = lax.iota(jnp.int32, LANES)
    zero16 = jnp.zeros((LANES,), jnp.float32)

    def build(j, p, pvec):
        crow = lax.shift_right_logical(j, 1)
        cbase = lax.shift_left(lax.bitwise_and(j, 1), 6)
        for g in range(GROUPS):
            sl = pl.ds(cbase + g * LANES, LANES)
            b16 = bi_v[crow, sl]
            a16 = ai_v[crow, sl]
            v16 = val_v[crow, sl]
            bidx3[p, pl.ds(g * LANES, LANES)] = lax.shift_right_logical(b16, 2)
            q16 = lax.shift_right_logical(a16 * 52348, 25)
            waddr16 = (a16 - q16 * INTER_DIM) * OUT_DIM
            colb16 = lax.shift_left(lax.bitwise_and(b16, PACK - 1), 5)
            for l in range(LANES):
                i = g * LANES + l
                waddr = waddr16[l]
                colb = colb16[l]
                vb = jnp.full((LANES,), v16[l], jnp.float32)
                w0 = w_v[pl.ds(waddr, LANES)]
                w1 = w_v[pl.ds(waddr + LANES, LANES)]
                rows3[p, i, pl.ds(colb, LANES)] = w0 * vb
                rows3[p, i, pl.ds(colb + LANES, LANES)] = w1 * vb
                for k in (32, 64, 96):
                    colz = lax.bitwise_and(colb + k, 127)
                    rows3[p, i, pl.ds(colz, LANES)] = zero16
                    rows3[p, i, pl.ds(colz + LANES, LANES)] = zero16

    def body(j, carry):
        p = lax.rem(j, 2)

        @pl.when(j >= 2)
        def _():
            pltpu.make_async_copy(
                rows3.at[p], acc.at[bidx3.at[p]], sems.at[p]).wait()

        @pl.when(j < NBLK)
        def _():
            build(j, p, jnp.full((LANES,), p, jnp.int32))
            pltpu.make_async_copy(
                rows3.at[p], acc.at[bidx3.at[p]], sems.at[p]).start(add=True)
        return carry

    lax.fori_loop(0, NBLK + 2, body, 0)
    plsc.subcore_barrier()
    pltpu.sync_copy(acc.at[pl.ds(sid * ROWS_PER_TEC, ROWS_PER_TEC)],
                    out_hbm.at[cid, pl.ds(sid * ROWS_PER_TEC, ROWS_PER_TEC)])


@jax.jit
def _sc_call(bi, ai, val, wflat, zeros):
    mesh = plsc.VectorSubcoreMesh(core_axis_name="c", subcore_axis_name="s")
    return pl.kernel(
        _sc_body,
        out_type=jax.ShapeDtypeStruct((NC, NPACK, 128), jnp.float32),
        mesh=mesh,
        scratch_types=[
            pltpu.VMEM_SHARED((NPACK, 128), jnp.float32),
            pltpu.VMEM((INTER_DIM * OUT_DIM,), jnp.float32),
            pltpu.VMEM((CROWS, 128), jnp.int32),
            pltpu.VMEM((CROWS, 128), jnp.int32),
            pltpu.VMEM((CROWS, 128), jnp.float32),
            pltpu.VMEM((2, BLK, 128), jnp.float32),
            pltpu.VMEM((2, BLK), jnp.int32),
            pltpu.SemaphoreType.DMA((2,)),
        ],
        compiler_params=pltpu.CompilerParams(needs_layout_passes=False),
    )(bi, ai, val, wflat, zeros)


def _add_body(p_ref, o_ref):
    o_ref[...] = p_ref[0] + p_ref[1]


@jax.jit
def _tc_add(partials):
    summed = pl.pallas_call(
        _add_body,
        out_shape=jax.ShapeDtypeStruct((NPACK, 128), jnp.float32),
    )(partials)
    return summed.reshape(N, OUT_DIM)


def kernel(batch_idx, active_idx, values, f, weights):
    del f
    bi = batch_idx.astype(jnp.int32).reshape(NNZ // 128, 128)
    ai = active_idx.astype(jnp.int32).reshape(NNZ // 128, 128)
    val = values.reshape(NNZ // 128, 128)
    wflat = weights.reshape(-1)
    zeros = jnp.zeros((NPACK, 128), jnp.float32)
    partials = _sc_call(bi, ai, val, wflat, zeros)
    return _tc_add(partials)
